# Initial kernel scaffold; baseline (speedup 1.0000x reference)
import jax
import jax.numpy as jnp
from jax.experimental import pallas as pl


def kernel(x_sta, x_ap, edge_index_r1, edge_attr_r1, edge_index_r2, edge_attr_r2, params):
    raise NotImplementedError("write your pallas kernel here")


# speedup vs baseline: 5.6818x; 5.6818x over previous
import functools

import jax
import jax.numpy as jnp
from jax import lax
from jax.experimental import pallas as pl
from jax.experimental.pallas import tpu as pltpu
from jax.experimental.pallas import tpu_sc as plsc

_N_STA = 10000
_N_AP = 10000
_E = 320000
_D = 128
_EPS = 1e-5

_NC = 2
_NS = 16
_NW = _NC * _NS
_EW = _E // _NW
_C = 128
_NCHUNK = _EW // _C
_TAIL = _EW - _NCHUNK * _C
_NP = 10240
_COLS = _NP // _NS

_mesh = plsc.VectorSubcoreMesh(core_axis_name="c", subcore_axis_name="s")
_sc_params = pltpu.CompilerParams(needs_layout_passes=False)


def _shuf(v, idx):
    dn = lax.GatherDimensionNumbers(offset_dims=(), collapsed_slice_dims=(0,),
                                    start_index_map=(0,))
    return lax.gather(v, idx[:, None], dn, (1,),
                      mode=lax.GatherScatterMode.PROMISE_IN_BOUNDS)


def _mm_kernel(x_ref, w_ref, b_ref, o_ref):
    o_ref[...] = (jnp.dot(x_ref[...], w_ref[...],
                          preferred_element_type=jnp.float32) + b_ref[...])


def _mm(x, w, b, bm):
    m, k = x.shape
    n = w.shape[1]
    return pl.pallas_call(
        _mm_kernel,
        grid=(m // bm,),
        in_specs=[pl.BlockSpec((bm, k), lambda i: (i, 0)),
                  pl.BlockSpec((k, n), lambda i: (0, 0)),
                  pl.BlockSpec((1, n), lambda i: (0, 0))],
        out_specs=pl.BlockSpec((bm, n), lambda i: (i, 0)),
        out_shape=jax.ShapeDtypeStruct((m, n), jnp.float32),
    )(x, w, b.reshape(1, n))


def _mm_stats_kernel(x_ref, w_ref, b_ref, o_ref, s_ref, q_ref):
    y = (jnp.dot(x_ref[...], w_ref[...],
                 preferred_element_type=jnp.float32) + b_ref[...])
    o_ref[...] = y

    @pl.when(pl.program_id(0) == 0)
    def _():
        s_ref[...] = jnp.zeros_like(s_ref)
        q_ref[...] = jnp.zeros_like(q_ref)

    s_ref[...] += jnp.sum(y, axis=0, keepdims=True)
    q_ref[...] += jnp.sum(y * y, axis=0, keepdims=True)


def _mm_stats(x, w, b, bm):
    m, k = x.shape
    n = w.shape[1]
    return pl.pallas_call(
        _mm_stats_kernel,
        grid=(m // bm,),
        in_specs=[pl.BlockSpec((bm, k), lambda i: (i, 0)),
                  pl.BlockSpec((k, n), lambda i: (0, 0)),
                  pl.BlockSpec((1, n), lambda i: (0, 0))],
        out_specs=[pl.BlockSpec((bm, n), lambda i: (i, 0)),
                   pl.BlockSpec((1, n), lambda i: (0, 0)),
                   pl.BlockSpec((1, n), lambda i: (0, 0))],
        out_shape=[jax.ShapeDtypeStruct((m, n), jnp.float32),
                   jax.ShapeDtypeStruct((1, n), jnp.float32),
                   jax.ShapeDtypeStruct((1, n), jnp.float32)],
    )(x, w, b.reshape(1, n))


def _mm3_kernel(x_ref, w0, b0, w1, b1, w2, b2, o0, o1, o2):
    x = x_ref[...]
    o0[...] = jnp.dot(x, w0[...], preferred_element_type=jnp.float32) + b0[...]
    o1[...] = jnp.dot(x, w1[...], preferred_element_type=jnp.float32) + b1[...]
    o2[...] = jnp.dot(x, w2[...], preferred_element_type=jnp.float32) + b2[...]


def _mm3(x, wb, bm):
    m, k = x.shape
    n = wb[0][0].shape[1]
    blk = pl.BlockSpec((bm, n), lambda i: (i, 0))
    wspec = pl.BlockSpec((k, n), lambda i: (0, 0))
    bspec = pl.BlockSpec((1, n), lambda i: (0, 0))
    args = [x]
    for w, b in wb:
        args += [w, b.reshape(1, n)]
    return pl.pallas_call(
        _mm3_kernel,
        grid=(m // bm,),
        in_specs=[pl.BlockSpec((bm, k), lambda i: (i, 0))]
        + [wspec, bspec] * 3,
        out_specs=[blk] * 3,
        out_shape=[jax.ShapeDtypeStruct((m, n), jnp.float32)] * 3,
    )(*args)


def _add2_stats_kernel(x_ref, o_ref, s_ref, q_ref):
    y = x_ref[0] + x_ref[1]
    o_ref[...] = y

    @pl.when(pl.program_id(0) == 0)
    def _():
        s_ref[...] = jnp.zeros_like(s_ref)
        q_ref[...] = jnp.zeros_like(q_ref)

    s_ref[...] += jnp.sum(y, axis=0, keepdims=True)
    q_ref[...] += jnp.sum(y * y, axis=0, keepdims=True)


def _add2_stats(parts, bm):
    _, m, n = parts.shape
    return pl.pallas_call(
        _add2_stats_kernel,
        grid=(m // bm,),
        in_specs=[pl.BlockSpec((2, bm, n), lambda i: (0, i, 0))],
        out_specs=[pl.BlockSpec((bm, n), lambda i: (i, 0)),
                   pl.BlockSpec((1, n), lambda i: (0, 0)),
                   pl.BlockSpec((1, n), lambda i: (0, 0))],
        out_shape=[jax.ShapeDtypeStruct((m, n), jnp.float32),
                   jax.ShapeDtypeStruct((1, n), jnp.float32),
                   jax.ShapeDtypeStruct((1, n), jnp.float32)],
    )(parts)


def _pred_kernel(x_ref, w1, b1, w2, b2, o_ref):
    h = x_ref[0] + x_ref[1]
    h = jnp.maximum(
        jnp.dot(h, w1[...], preferred_element_type=jnp.float32) + b1[...], 0.0)
    o_ref[...] = jnp.dot(h, w2[...], preferred_element_type=jnp.float32) + b2[...]


def _pred(parts, w1, b1, w2p, b2p, bm):
    _, m, n = parts.shape
    wspec = pl.BlockSpec((n, n), lambda i: (0, 0))
    bspec = pl.BlockSpec((1, n), lambda i: (0, 0))
    return pl.pallas_call(
        _pred_kernel,
        grid=(m // bm,),
        in_specs=[pl.BlockSpec((2, bm, n), lambda i: (0, i, 0)),
                  wspec, bspec, wspec, bspec],
        out_specs=pl.BlockSpec((bm, n), lambda i: (i, 0)),
        out_shape=jax.ShapeDtypeStruct((m, n), jnp.float32),
    )(parts, w1, b1.reshape(1, n), w2p, b2p)


@functools.partial(
    pl.kernel,
    out_type=(
        jax.ShapeDtypeStruct((_E,), jnp.float32),
        jax.ShapeDtypeStruct((_NC, _NP), jnp.float32),
    ),
    mesh=_mesh,
    compiler_params=_sc_params,
    scratch_types=dict(
        idx_s=pltpu.VMEM((_C,), jnp.int32),
        idx_d=pltpu.VMEM((_C,), jnp.int32),
        ai_v=pltpu.VMEM((_C, _D), jnp.float32),
        aj_v=pltpu.VMEM((_C, _D), jnp.float32),
        g_v=pltpu.VMEM((_C, _D), jnp.float32),
        attn_v=pltpu.VMEM((_D,), jnp.float32),
        pbuf=pltpu.VMEM((_C,), jnp.float32),
        stab=pltpu.VMEM((_NP,), jnp.float32),
        acc_c=pltpu.VMEM((_COLS,), jnp.float32),
        tmp_c=pltpu.VMEM((_COLS,), jnp.float32),
        s_sh=pltpu.VMEM_SHARED((_NS, _NP), jnp.float32),
        sem=pltpu.SemaphoreType.DMA,
    ),
)
def _sc_alpha(ai_hbm, aj_hbm, g_hbm, src_hbm, dst_hbm, attn_hbm,
              p_hbm, s_hbm,
              idx_s, idx_d, ai_v, aj_v, g_v, attn_v, pbuf, stab,
              acc_c, tmp_c, s_sh, sem):
    cid = lax.axis_index("c")
    sid = lax.axis_index("s")
    base = (sid * _NC + cid) * _EW

    pltpu.sync_copy(attn_hbm, attn_v)
    z16 = jnp.zeros((16,), jnp.float32)

    def zero_body(i, _):
        stab[pl.ds(i * 16, 16)] = z16
        return 0

    lax.fori_loop(0, _NP // 16, zero_body, 0)

    lane = lax.iota(jnp.int32, 16)

    def bsum(acc):
        for s in (8, 4, 2, 1):
            acc = acc + _shuf(acc, lane ^ s)
        return acc

    def group(g, nothing):
        av = jnp.zeros((16,), jnp.float32)
        for e16 in range(16):
            e = g * 16 + e16
            acc = jnp.zeros((16,), jnp.float32)
            for t in range(_D // 16):
                sl = pl.ds(t * 16, 16)
                v = ai_v[e, sl] + aj_v[e, sl] + g_v[e, sl]
                v = jnp.where(v > 0, v, 0.01 * v)
                acc = acc + v * attn_v[sl]
            av = jnp.where(lane == e16, bsum(acc), av)
        pv = jnp.exp(av)
        sl = pl.ds(g * 16, 16)
        pbuf[sl] = pv
        plsc.addupdate_scatter(stab, [idx_d[sl]], pv)
        return nothing

    def do_chunk(off, ce):
        pltpu.sync_copy(src_hbm.at[pl.ds(off, ce)], idx_s.at[pl.ds(0, ce)])
        pltpu.sync_copy(dst_hbm.at[pl.ds(off, ce)], idx_d.at[pl.ds(0, ce)])
        if ce == _C:
            pltpu.async_copy(ai_hbm.at[idx_d], ai_v, sem).wait()
            pltpu.async_copy(aj_hbm.at[idx_s], aj_v, sem).wait()
        else:
            pltpu.async_copy(ai_hbm.at[idx_d.at[pl.ds(0, ce)]],
                             ai_v.at[pl.ds(0, ce)], sem).wait()
            pltpu.async_copy(aj_hbm.at[idx_s.at[pl.ds(0, ce)]],
                             aj_v.at[pl.ds(0, ce)], sem).wait()
        pltpu.sync_copy(g_hbm.at[pl.ds(off, ce)], g_v.at[pl.ds(0, ce)])
        lax.fori_loop(0, ce // 16, group, 0)
        pltpu.sync_copy(pbuf.at[pl.ds(0, ce)], p_hbm.at[pl.ds(off, ce)])

    def chunk_body(ci, _):
        do_chunk(base + ci * _C, _C)
        return 0

    lax.fori_loop(0, _NCHUNK, chunk_body, 0)
    do_chunk(base + _NCHUNK * _C, _TAIL)

    pltpu.sync_copy(stab, s_sh.at[sid])
    plsc.subcore_barrier()
    cols = pl.ds(sid * _COLS, _COLS)
    pltpu.sync_copy(s_sh.at[0, cols], acc_c)

    def merge_body(k, _):
        pltpu.sync_copy(s_sh.at[k, cols], tmp_c)
        for t in range(_COLS // 16):
            sl = pl.ds(t * 16, 16)
            acc_c[sl] = acc_c[sl] + tmp_c[sl]
        return 0

    lax.fori_loop(1, _NS, merge_body, 0)
    pltpu.sync_copy(acc_c, s_hbm.at[cid, cols])


_RB = 128


@functools.partial(
    pl.kernel,
    out_type=jax.ShapeDtypeStruct((_NC, _NP, _D), jnp.float32),
    mesh=_mesh,
    compiler_params=_sc_params,
    scratch_types=dict(
        idx_s=pltpu.VMEM((_C,), jnp.int32),
        idx_d=pltpu.VMEM((_C,), jnp.int32),
        rows_v=pltpu.VMEM((_C, _D), jnp.float32),
        pbuf=pltpu.VMEM((_C,), jnp.float32),
        abuf=pltpu.VMEM((_C,), jnp.float32),
        s_tot=pltpu.VMEM((_NP,), jnp.float32),
        s_chk=pltpu.VMEM((_COLS,), jnp.float32),
        acc_sh=pltpu.VMEM_SHARED((_NP, _D), jnp.float32),
        sem=pltpu.SemaphoreType.DMA,
    ),
)
def _sc_aggregate(p_hbm, s_hbm, mj_hbm, src_hbm, dst_hbm,
                  out_hbm,
                  idx_s, idx_d, rows_v, pbuf, abuf, s_tot, s_chk,
                  acc_sh, sem):
    cid = lax.axis_index("c")
    sid = lax.axis_index("s")
    base = (sid * _NC + cid) * _EW

    pltpu.sync_copy(s_hbm.at[0], s_tot)

    def add_body(k, _):
        pltpu.sync_copy(s_hbm.at[1, pl.ds(k * _COLS, _COLS)], s_chk)
        for t in range(_COLS // 16):
            sl = pl.ds(k * _COLS + t * 16, 16)
            s_tot[sl] = s_tot[sl] + s_chk[pl.ds(t * 16, 16)]
        return 0

    lax.fori_loop(0, _NP // _COLS, add_body, 0)

    z16 = jnp.zeros((16,), jnp.float32)

    def zrow_body(r, _):
        for t in range(_D // 16):
            rows_v[r, pl.ds(t * 16, 16)] = z16
        return 0

    lax.fori_loop(0, _RB, zrow_body, 0)
    row0 = sid * (_NP // _NS)
    for i in range(5):
        pltpu.sync_copy(rows_v, acc_sh.at[pl.ds(row0 + i * _RB, _RB)])
    plsc.subcore_barrier()

    lane = lax.iota(jnp.int32, 16)

    def group(g, nothing):
        sl = pl.ds(g * 16, 16)
        sg = plsc.load_gather(s_tot, [idx_d[sl]])
        av = pbuf[sl] / (sg + 1e-16)
        abuf[sl] = av
        for e16 in range(16):
            e = g * 16 + e16
            sv = _shuf(av, jnp.full((16,), e16, jnp.int32))
            for t in range(_D // 16):
                tsl = pl.ds(t * 16, 16)
                rows_v[e, tsl] = rows_v[e, tsl] * sv
        return nothing

    def do_chunk(off, ce):
        pltpu.sync_copy(src_hbm.at[pl.ds(off, ce)], idx_s.at[pl.ds(0, ce)])
        pltpu.sync_copy(dst_hbm.at[pl.ds(off, ce)], idx_d.at[pl.ds(0, ce)])
        pltpu.sync_copy(p_hbm.at[pl.ds(off, ce)], pbuf.at[pl.ds(0, ce)])
        if ce == _C:
            pltpu.async_copy(mj_hbm.at[idx_s], rows_v, sem).wait()
            lax.fori_loop(0, ce // 16, group, 0)
            pltpu.sync_copy(rows_v, acc_sh.at[idx_d], add=True)
        else:
            pltpu.async_copy(mj_hbm.at[idx_s.at[pl.ds(0, ce)]],
                             rows_v.at[pl.ds(0, ce)], sem).wait()
            lax.fori_loop(0, ce // 16, group, 0)
            pltpu.sync_copy(rows_v.at[pl.ds(0, ce)],
                            acc_sh.at[idx_d.at[pl.ds(0, ce)]], add=True)

    def chunk_body(ci, _):
        do_chunk(base + ci * _C, _C)
        return 0

    lax.fori_loop(0, _NCHUNK, chunk_body, 0)
    do_chunk(base + _NCHUNK * _C, _TAIL)

    plsc.subcore_barrier()
    for i in range(5):
        rows = pl.ds(row0 + i * _RB, _RB)
        pltpu.sync_copy(acc_sh.at[rows], rows_v)
        pltpu.sync_copy(rows_v, out_hbm.at[cid, rows])


def _affine(g, b, ssum, ssq, m):
    mu = ssum.reshape(-1) / m
    var = ssq.reshape(-1) / m - mu * mu
    a = g / jnp.sqrt(var + _EPS)
    return a, b - a * mu


def _fold(a, c, w, b):
    return a[:, None] * w, b + c @ w


def kernel(x_sta, x_ap, edge_index_r1, edge_attr_r1, edge_index_r2,
           edge_attr_r2, params):
    p = params
    src1 = edge_index_r1[0].astype(jnp.int32)
    dst1 = edge_index_r1[1].astype(jnp.int32)
    src2 = edge_index_r2[0].astype(jnp.int32)
    dst2 = edge_index_r2[1].astype(jnp.int32)

    h_sta, s_sum, s_sq = _mm_stats(x_sta, p["in_sta_W"], p["in_sta_b"], 1000)
    h_ap, a_sum, a_sq = _mm_stats(x_ap, p["in_ap_W"], p["in_ap_b"], 1000)
    e1, e1_sum, e1_sq = _mm_stats(edge_attr_r1, p["ein_r1_W"], p["ein_r1_b"], 2000)
    e2, e2_sum, e2_sq = _mm_stats(edge_attr_r2, p["ein_r2_W"], p["ein_r2_b"], 2000)

    e1_mu = e1_sum.reshape(-1) / _E
    e1_var = e1_sq.reshape(-1) / _E - e1_mu * e1_mu
    e2_mu = e2_sum.reshape(-1) / _E
    e2_var = e2_sq.reshape(-1) / _E - e2_mu * e2_mu

    def edge_affines(mu0, var0):
        out = []
        a_tot = jnp.ones((_D,), jnp.float32)
        c_tot = jnp.zeros((_D,), jnp.float32)
        mu, var = mu0, var0
        for l in range(2):
            g = p["l%d_en_g" % l]
            b = p["l%d_en_b" % l]
            a = g / jnp.sqrt(var + _EPS)
            c = b - a * mu
            a_tot, c_tot = a * a_tot, a * c_tot + c
            out.append((a_tot, c_tot))
            mu, var = b, a * a * var
        return out

    e1_aff = edge_affines(e1_mu, e1_var)
    e2_aff = edge_affines(e2_mu, e2_var)

    node_stats = ((s_sum, s_sq), (a_sum, a_sq))
    out = None
    for l in range(2):
        g_nn, b_nn = p["l%d_nn_g" % l], p["l%d_nn_b" % l]
        sta_a, sta_c = _affine(g_nn, b_nn, *node_stats[0], _N_STA)
        ap_a, ap_c = _affine(g_nn, b_nn, *node_stats[1], _N_AP)
        pr1, pr2 = "l%d_r1" % l, "l%d_r2" % l

        wb_sta = [
            _fold(sta_a, sta_c, p[pr1 + "_ni_W"], p[pr1 + "_ni_b"]),
            _fold(sta_a, sta_c, p[pr2 + "_nj_W"], p[pr2 + "_nj_b"]),
            _fold(sta_a, sta_c, p[pr2 + "_node_W"], p[pr2 + "_node_b"]),
        ]
        wb_ap = [
            _fold(ap_a, ap_c, p[pr2 + "_ni_W"], p[pr2 + "_ni_b"]),
            _fold(ap_a, ap_c, p[pr1 + "_nj_W"], p[pr1 + "_nj_b"]),
            _fold(ap_a, ap_c, p[pr1 + "_node_W"], p[pr1 + "_node_b"]),
        ]
        bm_n = 1000 if l == 0 else 1280
        ai1, aj2, mj2 = _mm3(h_sta, wb_sta, bm_n)
        ai2, aj1, mj1 = _mm3(h_ap, wb_ap, bm_n)

        w_f1, b_f1 = _fold(*e1_aff[l], p[pr1 + "_fij_W"], p[pr1 + "_fij_b"])
        w_f2, b_f2 = _fold(*e2_aff[l], p[pr2 + "_fij_W"], p[pr2 + "_fij_b"])
        g1 = _mm(e1, w_f1, b_f1, 2000)
        g2 = _mm(e2, w_f2, b_f2, 2000)

        p1, seg1 = _sc_alpha(ai1, aj1, g1, src1, dst1,
                             p[pr1 + "_attn"].reshape(-1))
        part1 = _sc_aggregate(p1, seg1, mj1, src1, dst1)
        p2, seg2 = _sc_alpha(ai2, aj2, g2, src2, dst2,
                             p[pr2 + "_attn"].reshape(-1))
        part2 = _sc_aggregate(p2, seg2, mj2, src2, dst2)

        if l == 0:
            h_sta, s_sum, s_sq = _add2_stats(part1, 1280)
            h_ap, a_sum, a_sq = _add2_stats(part2, 1280)
            node_stats = ((s_sum, s_sq), (a_sum, a_sq))
        else:
            w2p = jnp.pad(p["pred_W2"], ((0, 0), (0, _D - 1)))
            b2p = jnp.pad(p["pred_b2"], (0, _D - 1)).reshape(1, _D)
            out = _pred(part1, p["pred_W1"], p["pred_b1"], w2p, b2p, 1280)
    return out[:_N_STA, :1]


# speedup vs baseline: 6.2844x; 1.1061x over previous
import functools

import jax
import jax.numpy as jnp
from jax import lax
from jax.experimental import pallas as pl
from jax.experimental.pallas import tpu as pltpu
from jax.experimental.pallas import tpu_sc as plsc

_N_STA = 10000
_N_AP = 10000
_E = 320000
_D = 128
_EPS = 1e-5

_NC = 2
_NS = 16
_EW = _E // _NS
_C = 128
_NCHUNK = _EW // _C
_TAIL = _EW - _NCHUNK * _C
_NP = 10240
_COLS = _NP // _NS
_RB = 128

_mesh = plsc.VectorSubcoreMesh(core_axis_name="c", subcore_axis_name="s")
_sc_params = pltpu.CompilerParams(needs_layout_passes=False)


def _shuf(v, idx):
    dn = lax.GatherDimensionNumbers(offset_dims=(), collapsed_slice_dims=(0,),
                                    start_index_map=(0,))
    return lax.gather(v, idx[:, None], dn, (1,),
                      mode=lax.GatherScatterMode.PROMISE_IN_BOUNDS)


def _dot(x, w):
    return jnp.dot(x, w, preferred_element_type=jnp.float32,
                   precision=lax.Precision.HIGHEST)


def _mm_kernel(x_ref, w_ref, b_ref, o_ref):
    o_ref[...] = _dot(x_ref[...], w_ref[...]) + b_ref[...]


def _mm(x, w, b, bm):
    m, k = x.shape
    n = w.shape[1]
    return pl.pallas_call(
        _mm_kernel,
        grid=(m // bm,),
        in_specs=[pl.BlockSpec((bm, k), lambda i: (i, 0)),
                  pl.BlockSpec((k, n), lambda i: (0, 0)),
                  pl.BlockSpec((1, n), lambda i: (0, 0))],
        out_specs=pl.BlockSpec((bm, n), lambda i: (i, 0)),
        out_shape=jax.ShapeDtypeStruct((m, n), jnp.float32),
    )(x, w, b.reshape(1, n))


def _mm_stats_kernel(x_ref, w_ref, b_ref, o_ref, s_ref, q_ref):
    y = _dot(x_ref[...], w_ref[...]) + b_ref[...]
    o_ref[...] = y

    @pl.when(pl.program_id(0) == 0)
    def _():
        s_ref[...] = jnp.zeros_like(s_ref)
        q_ref[...] = jnp.zeros_like(q_ref)

    s_ref[...] += jnp.sum(y, axis=0, keepdims=True)
    q_ref[...] += jnp.sum(y * y, axis=0, keepdims=True)


def _mm_stats(x, w, b, bm):
    m, k = x.shape
    n = w.shape[1]
    return pl.pallas_call(
        _mm_stats_kernel,
        grid=(m // bm,),
        in_specs=[pl.BlockSpec((bm, k), lambda i: (i, 0)),
                  pl.BlockSpec((k, n), lambda i: (0, 0)),
                  pl.BlockSpec((1, n), lambda i: (0, 0))],
        out_specs=[pl.BlockSpec((bm, n), lambda i: (i, 0)),
                   pl.BlockSpec((1, n), lambda i: (0, 0)),
                   pl.BlockSpec((1, n), lambda i: (0, 0))],
        out_shape=[jax.ShapeDtypeStruct((m, n), jnp.float32),
                   jax.ShapeDtypeStruct((1, n), jnp.float32),
                   jax.ShapeDtypeStruct((1, n), jnp.float32)],
    )(x, w, b.reshape(1, n))


def _mm3_kernel2d(x_ref, w0, b0, w1, b1, w2, b2, o0, o1, o2):
    x = x_ref[...]
    o0[...] = _dot(x, w0[...]) + b0[...]
    o1[...] = _dot(x, w1[...]) + b1[...]
    o2[...] = _dot(x, w2[...]) + b2[...]


def _mm3_kernel3d(x_ref, w0, b0, w1, b1, w2, b2, o0, o1, o2):
    x = x_ref[0]
    o0[...] = _dot(x, w0[...]) + b0[...]
    o1[...] = _dot(x, w1[...]) + b1[...]
    o2[...] = _dot(x, w2[...]) + b2[...]


def _mm3(x, row, wb, bm):
    n = wb[0][0].shape[1]
    if x.ndim == 2:
        m, k = x.shape
        xspec = pl.BlockSpec((bm, k), lambda i: (i, 0))
        body = _mm3_kernel2d
    else:
        _, m, k = x.shape
        xspec = pl.BlockSpec((1, bm, k), lambda i: (row, i, 0))
        body = _mm3_kernel3d
    blk = pl.BlockSpec((bm, n), lambda i: (i, 0))
    wspec = pl.BlockSpec((k, n), lambda i: (0, 0))
    bspec = pl.BlockSpec((1, n), lambda i: (0, 0))
    args = [x]
    for w, b in wb:
        args += [w, b.reshape(1, n)]
    return pl.pallas_call(
        body,
        grid=(m // bm,),
        in_specs=[xspec] + [wspec, bspec] * 3,
        out_specs=[blk] * 3,
        out_shape=[jax.ShapeDtypeStruct((m, n), jnp.float32)] * 3,
    )(*args)


def _stats_kernel(x_ref, s_ref, q_ref):
    y = x_ref[0]

    @pl.when(pl.program_id(0) == 0)
    def _():
        s_ref[...] = jnp.zeros_like(s_ref)
        q_ref[...] = jnp.zeros_like(q_ref)

    s_ref[...] += jnp.sum(y, axis=0, keepdims=True)
    q_ref[...] += jnp.sum(y * y, axis=0, keepdims=True)


def _stats(parts, row, bm):
    _, m, n = parts.shape
    return pl.pallas_call(
        _stats_kernel,
        grid=(m // bm,),
        in_specs=[pl.BlockSpec((1, bm, n), lambda i: (row, i, 0))],
        out_specs=[pl.BlockSpec((1, n), lambda i: (0, 0)),
                   pl.BlockSpec((1, n), lambda i: (0, 0))],
        out_shape=[jax.ShapeDtypeStruct((1, n), jnp.float32),
                   jax.ShapeDtypeStruct((1, n), jnp.float32)],
    )(parts)


def _pred_kernel(x_ref, w1, b1, w2, b2, o_ref):
    h = jnp.maximum(_dot(x_ref[0], w1[...]) + b1[...], 0.0)
    o_ref[...] = _dot(h, w2[...]) + b2[...]


def _pred(parts, w1, b1, w2p, b2p, bm):
    _, m, n = parts.shape
    wspec = pl.BlockSpec((n, n), lambda i: (0, 0))
    bspec = pl.BlockSpec((1, n), lambda i: (0, 0))
    return pl.pallas_call(
        _pred_kernel,
        grid=(m // bm,),
        in_specs=[pl.BlockSpec((1, bm, n), lambda i: (0, i, 0)),
                  wspec, bspec, wspec, bspec],
        out_specs=pl.BlockSpec((bm, n), lambda i: (i, 0)),
        out_shape=jax.ShapeDtypeStruct((m, n), jnp.float32),
    )(parts, w1, b1.reshape(1, n), w2p, b2p)


@functools.partial(
    pl.kernel,
    out_type=(
        jax.ShapeDtypeStruct((_E,), jnp.float32),
        jax.ShapeDtypeStruct((_E,), jnp.float32),
        jax.ShapeDtypeStruct((_NC, _NP), jnp.float32),
    ),
    mesh=_mesh,
    compiler_params=_sc_params,
    scratch_types=dict(
        idx_s0=pltpu.VMEM((_C,), jnp.int32),
        idx_d0=pltpu.VMEM((_C,), jnp.int32),
        ai_v0=pltpu.VMEM((_C, _D), jnp.float32),
        aj_v0=pltpu.VMEM((_C, _D), jnp.float32),
        g_v0=pltpu.VMEM((_C, _D), jnp.float32),
        sem0=pltpu.SemaphoreType.DMA,
        idx_s1=pltpu.VMEM((_C,), jnp.int32),
        idx_d1=pltpu.VMEM((_C,), jnp.int32),
        ai_v1=pltpu.VMEM((_C, _D), jnp.float32),
        aj_v1=pltpu.VMEM((_C, _D), jnp.float32),
        g_v1=pltpu.VMEM((_C, _D), jnp.float32),
        sem1=pltpu.SemaphoreType.DMA,
        attn_v=pltpu.VMEM((_D,), jnp.float32),
        pbuf=pltpu.VMEM((_C,), jnp.float32),
        stab=pltpu.VMEM((_NP,), jnp.float32),
        acc_c=pltpu.VMEM((_COLS,), jnp.float32),
        tmp_c=pltpu.VMEM((_COLS,), jnp.float32),
        s_sh=pltpu.VMEM_SHARED((_NS, _NP), jnp.float32),
    ),
)
def _sc_alpha(ai1, aj1, g1, src1, dst1, attn1,
              ai2, aj2, g2, src2, dst2, attn2,
              p1_hbm, p2_hbm, s_hbm,
              idx_s0, idx_d0, ai_v0, aj_v0, g_v0, sem0,
              idx_s1, idx_d1, ai_v1, aj_v1, g_v1, sem1,
              attn_v, pbuf, stab, acc_c, tmp_c, s_sh):
    cid = lax.axis_index("c")
    sid = lax.axis_index("s")
    base = sid * _EW

    z16 = jnp.zeros((16,), jnp.float32)

    def zero_body(i, _):
        stab[pl.ds(i * 16, 16)] = z16
        return 0

    lax.fori_loop(0, _NP // 16, zero_body, 0)

    lane = lax.iota(jnp.int32, 16)

    def bsum(acc):
        for s in (8, 4, 2, 1):
            acc = acc + _shuf(acc, lane ^ s)
        return acc

    bufs = ((idx_s0, idx_d0, ai_v0, aj_v0, g_v0, sem0),
            (idx_s1, idx_d1, ai_v1, aj_v1, g_v1, sem1))

    def run_relation(ai_hbm, aj_hbm, g_hbm, src_hbm, dst_hbm, attn_hbm, p_hbm):
        pltpu.sync_copy(attn_hbm, attn_v)

        def issue(b, off):
            ids, idd, av, jv, gv, sem = bufs[b]
            pltpu.sync_copy(src_hbm.at[pl.ds(off, _C)], ids)
            pltpu.sync_copy(dst_hbm.at[pl.ds(off, _C)], idd)
            pltpu.async_copy(ai_hbm.at[idd], av, sem)
            pltpu.async_copy(aj_hbm.at[ids], jv, sem)
            pltpu.async_copy(g_hbm.at[pl.ds(off, _C)], gv, sem)

        def wait(b):
            ids, idd, av, jv, gv, sem = bufs[b]
            pltpu.make_async_copy(ai_hbm.at[idd], av, sem).wait()
            pltpu.make_async_copy(aj_hbm.at[ids], jv, sem).wait()
            pltpu.make_async_copy(g_hbm.at[pl.ds(0, _C)], gv, sem).wait()

        def compute(b, off, ce):
            ids, idd, av, jv, gv, sem = bufs[b]

            def group(g, nothing):
                alv = jnp.zeros((16,), jnp.float32)
                for e16 in range(16):
                    e = g * 16 + e16
                    acc = jnp.zeros((16,), jnp.float32)
                    for t in range(_D // 16):
                        sl = pl.ds(t * 16, 16)
                        v = av[e, sl] + jv[e, sl] + gv[e, sl]
                        v = jnp.where(v > 0, v, 0.01 * v)
                        acc = acc + v * attn_v[sl]
                    alv = jnp.where(lane == e16, bsum(acc), alv)
                pv = jnp.exp(alv)
                sl = pl.ds(g * 16, 16)
                pbuf[sl] = pv
                plsc.addupdate_scatter(stab, [idd[sl]], pv)
                return nothing

            lax.fori_loop(0, ce // 16, group, 0)
            pltpu.sync_copy(pbuf.at[pl.ds(0, ce)],
                            p_hbm.at[pl.ds(off, ce)])

        issue(0, base)

        def pair(pi, _):
            off0 = base + (2 * pi) * _C
            issue(1, off0 + _C)
            wait(0)
            compute(0, off0, _C)

            @pl.when(pi < _NCHUNK // 2 - 1)
            def _():
                issue(0, off0 + 2 * _C)

            wait(1)
            compute(1, off0 + _C, _C)
            return 0

        lax.fori_loop(0, _NCHUNK // 2, pair, 0)

        toff = base + _NCHUNK * _C
        pltpu.sync_copy(src_hbm.at[pl.ds(toff, _TAIL)],
                        idx_s0.at[pl.ds(0, _TAIL)])
        pltpu.sync_copy(dst_hbm.at[pl.ds(toff, _TAIL)],
                        idx_d0.at[pl.ds(0, _TAIL)])
        pltpu.async_copy(ai_hbm.at[idx_d0.at[pl.ds(0, _TAIL)]],
                         ai_v0.at[pl.ds(0, _TAIL)], sem0).wait()
        pltpu.async_copy(aj_hbm.at[idx_s0.at[pl.ds(0, _TAIL)]],
                         aj_v0.at[pl.ds(0, _TAIL)], sem0).wait()
        pltpu.sync_copy(g_hbm.at[pl.ds(toff, _TAIL)], g_v0.at[pl.ds(0, _TAIL)])
        compute(0, toff, _TAIL)

    @pl.when(cid == 0)
    def _():
        run_relation(ai1, aj1, g1, src1, dst1, attn1, p1_hbm)

    @pl.when(cid == 1)
    def _():
        run_relation(ai2, aj2, g2, src2, dst2, attn2, p2_hbm)

    pltpu.sync_copy(stab, s_sh.at[sid])
    plsc.subcore_barrier()
    cols = pl.ds(sid * _COLS, _COLS)
    pltpu.sync_copy(s_sh.at[0, cols], acc_c)

    def merge_body(k, _):
        pltpu.sync_copy(s_sh.at[k, cols], tmp_c)
        for t in range(_COLS // 16):
            sl = pl.ds(t * 16, 16)
            acc_c[sl] = acc_c[sl] + tmp_c[sl]
        return 0

    lax.fori_loop(1, _NS, merge_body, 0)
    pltpu.sync_copy(acc_c, s_hbm.at[cid, cols])


@functools.partial(
    pl.kernel,
    out_type=jax.ShapeDtypeStruct((_NC, _NP, _D), jnp.float32),
    mesh=_mesh,
    compiler_params=_sc_params,
    scratch_types=dict(
        idx_s0=pltpu.VMEM((_C,), jnp.int32),
        idx_d0=pltpu.VMEM((_C,), jnp.int32),
        rows0=pltpu.VMEM((_C, _D), jnp.float32),
        pb0=pltpu.VMEM((_C,), jnp.float32),
        sem0=pltpu.SemaphoreType.DMA,
        idx_s1=pltpu.VMEM((_C,), jnp.int32),
        idx_d1=pltpu.VMEM((_C,), jnp.int32),
        rows1=pltpu.VMEM((_C, _D), jnp.float32),
        pb1=pltpu.VMEM((_C,), jnp.float32),
        sem1=pltpu.SemaphoreType.DMA,
        s_tot=pltpu.VMEM((_NP,), jnp.float32),
        acc_sh=pltpu.VMEM_SHARED((_NP, _D), jnp.float32),
    ),
)
def _sc_aggregate(p1_hbm, p2_hbm, s_hbm, mj1, src1, dst1, mj2, src2, dst2,
                  out_hbm,
                  idx_s0, idx_d0, rows0, pb0, sem0,
                  idx_s1, idx_d1, rows1, pb1, sem1,
                  s_tot, acc_sh):
    cid = lax.axis_index("c")
    sid = lax.axis_index("s")
    base = sid * _EW

    pltpu.sync_copy(s_hbm.at[cid], s_tot)

    z16 = jnp.zeros((16,), jnp.float32)

    def zrow_body(r, _):
        for t in range(_D // 16):
            rows0[r, pl.ds(t * 16, 16)] = z16
        return 0

    lax.fori_loop(0, _RB, zrow_body, 0)
    row0 = sid * (_NP // _NS)
    for i in range(5):
        pltpu.sync_copy(rows0, acc_sh.at[pl.ds(row0 + i * _RB, _RB)])
    plsc.subcore_barrier()

    bufs = ((idx_s0, idx_d0, rows0, pb0, sem0),
            (idx_s1, idx_d1, rows1, pb1, sem1))

    def run_relation(mj_hbm, src_hbm, dst_hbm, p_hbm):
        def issue(b, off):
            ids, idd, rows, pb, sem = bufs[b]
            pltpu.sync_copy(src_hbm.at[pl.ds(off, _C)], ids)
            pltpu.sync_copy(dst_hbm.at[pl.ds(off, _C)], idd)
            pltpu.sync_copy(p_hbm.at[pl.ds(off, _C)], pb)
            pltpu.async_copy(mj_hbm.at[ids], rows, sem)

        def wait(b):
            ids, idd, rows, pb, sem = bufs[b]
            pltpu.make_async_copy(mj_hbm.at[ids], rows, sem).wait()

        def compute(b, ce):
            ids, idd, rows, pb, sem = bufs[b]

            def group(g, nothing):
                sl = pl.ds(g * 16, 16)
                sg = plsc.load_gather(s_tot, [idd[sl]])
                av = pb[sl] / (sg + 1e-16)
                for e16 in range(16):
                    e = g * 16 + e16
                    sv = _shuf(av, jnp.full((16,), e16, jnp.int32))
                    for t in range(_D // 16):
                        tsl = pl.ds(t * 16, 16)
                        rows[e, tsl] = rows[e, tsl] * sv
                return nothing

            lax.fori_loop(0, ce // 16, group, 0)
            if ce == _C:
                pltpu.sync_copy(rows, acc_sh.at[idd], add=True)
            else:
                pltpu.sync_copy(rows.at[pl.ds(0, ce)],
                                acc_sh.at[idd.at[pl.ds(0, ce)]], add=True)

        issue(0, base)

        def pair(pi, _):
            off0 = base + (2 * pi) * _C
            issue(1, off0 + _C)
            wait(0)
            compute(0, _C)

            @pl.when(pi < _NCHUNK // 2 - 1)
            def _():
                issue(0, off0 + 2 * _C)

            wait(1)
            compute(1, _C)
            return 0

        lax.fori_loop(0, _NCHUNK // 2, pair, 0)

        toff = base + _NCHUNK * _C
        pltpu.sync_copy(src_hbm.at[pl.ds(toff, _TAIL)],
                        idx_s0.at[pl.ds(0, _TAIL)])
        pltpu.sync_copy(dst_hbm.at[pl.ds(toff, _TAIL)],
                        idx_d0.at[pl.ds(0, _TAIL)])
        pltpu.sync_copy(p_hbm.at[pl.ds(toff, _TAIL)],
                        pb0.at[pl.ds(0, _TAIL)])
        pltpu.async_copy(mj_hbm.at[idx_s0.at[pl.ds(0, _TAIL)]],
                         rows0.at[pl.ds(0, _TAIL)], sem0).wait()
        compute(0, _TAIL)

    @pl.when(cid == 0)
    def _():
        run_relation(mj1, src1, dst1, p1_hbm)

    @pl.when(cid == 1)
    def _():
        run_relation(mj2, src2, dst2, p2_hbm)

    plsc.subcore_barrier()
    for i in range(5):
        rsl = pl.ds(row0 + i * _RB, _RB)
        pltpu.sync_copy(acc_sh.at[rsl], rows0)
        pltpu.sync_copy(rows0, out_hbm.at[cid, rsl])


def _affine(g, b, ssum, ssq, m):
    mu = ssum.reshape(-1) / m
    var = ssq.reshape(-1) / m - mu * mu
    a = g / jnp.sqrt(var + _EPS)
    return a, b - a * mu


def _fold(a, c, w, b):
    return a[:, None] * w, b + c @ w


def kernel(x_sta, x_ap, edge_index_r1, edge_attr_r1, edge_index_r2,
           edge_attr_r2, params):
    p = params
    src1 = edge_index_r1[0].astype(jnp.int32)
    dst1 = edge_index_r1[1].astype(jnp.int32)
    src2 = edge_index_r2[0].astype(jnp.int32)
    dst2 = edge_index_r2[1].astype(jnp.int32)

    h_sta, s_sum, s_sq = _mm_stats(x_sta, p["in_sta_W"], p["in_sta_b"], 1000)
    h_ap, a_sum, a_sq = _mm_stats(x_ap, p["in_ap_W"], p["in_ap_b"], 1000)
    e1, e1_sum, e1_sq = _mm_stats(edge_attr_r1, p["ein_r1_W"], p["ein_r1_b"], 2000)
    e2, e2_sum, e2_sq = _mm_stats(edge_attr_r2, p["ein_r2_W"], p["ein_r2_b"], 2000)

    e1_mu = e1_sum.reshape(-1) / _E
    e1_var = e1_sq.reshape(-1) / _E - e1_mu * e1_mu
    e2_mu = e2_sum.reshape(-1) / _E
    e2_var = e2_sq.reshape(-1) / _E - e2_mu * e2_mu

    def edge_affines(mu0, var0):
        out = []
        a_tot = jnp.ones((_D,), jnp.float32)
        c_tot = jnp.zeros((_D,), jnp.float32)
        mu, var = mu0, var0
        for l in range(2):
            g = p["l%d_en_g" % l]
            b = p["l%d_en_b" % l]
            a = g / jnp.sqrt(var + _EPS)
            c = b - a * mu
            a_tot, c_tot = a * a_tot, a * c_tot + c
            out.append((a_tot, c_tot))
            mu, var = b, a * a * var
        return out

    e1_aff = edge_affines(e1_mu, e1_var)
    e2_aff = edge_affines(e2_mu, e2_var)

    node_stats = ((s_sum, s_sq), (a_sum, a_sq))
    hs, ha = h_sta, h_ap
    hrow_s = hrow_a = 0
    out = None
    for l in range(2):
        g_nn, b_nn = p["l%d_nn_g" % l], p["l%d_nn_b" % l]
        sta_a, sta_c = _affine(g_nn, b_nn, *node_stats[0], _N_STA)
        ap_a, ap_c = _affine(g_nn, b_nn, *node_stats[1], _N_AP)
        pr1, pr2 = "l%d_r1" % l, "l%d_r2" % l

        wb_sta = [
            _fold(sta_a, sta_c, p[pr1 + "_ni_W"], p[pr1 + "_ni_b"]),
            _fold(sta_a, sta_c, p[pr2 + "_nj_W"], p[pr2 + "_nj_b"]),
            _fold(sta_a, sta_c, p[pr2 + "_node_W"], p[pr2 + "_node_b"]),
        ]
        wb_ap = [
            _fold(ap_a, ap_c, p[pr2 + "_ni_W"], p[pr2 + "_ni_b"]),
            _fold(ap_a, ap_c, p[pr1 + "_nj_W"], p[pr1 + "_nj_b"]),
            _fold(ap_a, ap_c, p[pr1 + "_node_W"], p[pr1 + "_node_b"]),
        ]
        bm_n = 1000 if l == 0 else 1280
        ai1, aj2, mj2 = _mm3(hs, hrow_s, wb_sta, bm_n)
        ai2, aj1, mj1 = _mm3(ha, hrow_a, wb_ap, bm_n)

        w_f1, b_f1 = _fold(*e1_aff[l], p[pr1 + "_fij_W"], p[pr1 + "_fij_b"])
        w_f2, b_f2 = _fold(*e2_aff[l], p[pr2 + "_fij_W"], p[pr2 + "_fij_b"])
        g1 = _mm(e1, w_f1, b_f1, 2000)
        g2 = _mm(e2, w_f2, b_f2, 2000)

        pe1, pe2, seg = _sc_alpha(ai1, aj1, g1, src1, dst1,
                            p[pr1 + "_attn"].reshape(-1),
                            ai2, aj2, g2, src2, dst2,
                            p[pr2 + "_attn"].reshape(-1))
        parts = _sc_aggregate(pe1, pe2, seg, mj1, src1, dst1, mj2, src2, dst2)

        if l == 0:
            s_sum, s_sq = _stats(parts, 0, 1280)
            a_sum, a_sq = _stats(parts, 1, 1280)
            node_stats = ((s_sum, s_sq), (a_sum, a_sq))
            hs = ha = parts
            hrow_s, hrow_a = 0, 1
        else:
            w2p = jnp.pad(p["pred_W2"], ((0, 0), (0, _D - 1)))
            b2p = jnp.pad(p["pred_b2"], (0, _D - 1)).reshape(1, _D)
            out = _pred(parts, p["pred_W1"], p["pred_b1"], w2p, b2p, 1280)
    return out[:_N_STA, :1]


# speedup vs baseline: 8.6546x; 1.3771x over previous
import functools

import jax
import jax.numpy as jnp
from jax import lax
from jax.experimental import pallas as pl
from jax.experimental.pallas import tpu as pltpu
from jax.experimental.pallas import tpu_sc as plsc

_N_STA = 10000
_N_AP = 10000
_E = 320000
_D = 128
_EPS = 1e-5

_NC = 2
_NS = 16
_NW = _NC * _NS
_EW = _E // _NW
_C = 128
_NCHUNK = _EW // _C
_TAIL = _EW - _NCHUNK * _C
_NP = 10240
_COLS = _NP // _NS

_mesh = plsc.VectorSubcoreMesh(core_axis_name="c", subcore_axis_name="s")
_sc_params = pltpu.CompilerParams(needs_layout_passes=False)


def _shuf(v, idx):
    dn = lax.GatherDimensionNumbers(offset_dims=(), collapsed_slice_dims=(0,),
                                    start_index_map=(0,))
    return lax.gather(v, idx[:, None], dn, (1,),
                      mode=lax.GatherScatterMode.PROMISE_IN_BOUNDS)


def _mm_kernel(x_ref, w_ref, b_ref, o_ref):
    o_ref[...] = (jnp.dot(x_ref[...], w_ref[...],
                          preferred_element_type=jnp.float32,
                          precision=lax.Precision.HIGHEST) + b_ref[...])


def _mm(x, w, b, bm):
    m, k = x.shape
    n = w.shape[1]
    return pl.pallas_call(
        _mm_kernel,
        grid=(m // bm,),
        in_specs=[pl.BlockSpec((bm, k), lambda i: (i, 0)),
                  pl.BlockSpec((k, n), lambda i: (0, 0)),
                  pl.BlockSpec((1, n), lambda i: (0, 0))],
        out_specs=pl.BlockSpec((bm, n), lambda i: (i, 0)),
        out_shape=jax.ShapeDtypeStruct((m, n), jnp.float32),
    )(x, w, b.reshape(1, n))


def _mm_stats_kernel(x_ref, w_ref, b_ref, o_ref, s_ref, q_ref):
    y = (jnp.dot(x_ref[...], w_ref[...],
                 preferred_element_type=jnp.float32,
                          precision=lax.Precision.HIGHEST) + b_ref[...])
    o_ref[...] = y

    @pl.when(pl.program_id(0) == 0)
    def _():
        s_ref[...] = jnp.zeros_like(s_ref)
        q_ref[...] = jnp.zeros_like(q_ref)

    s_ref[...] += jnp.sum(y, axis=0, keepdims=True)
    q_ref[...] += jnp.sum(y * y, axis=0, keepdims=True)


def _mm_stats(x, w, b, bm):
    m, k = x.shape
    n = w.shape[1]
    return pl.pallas_call(
        _mm_stats_kernel,
        grid=(m // bm,),
        in_specs=[pl.BlockSpec((bm, k), lambda i: (i, 0)),
                  pl.BlockSpec((k, n), lambda i: (0, 0)),
                  pl.BlockSpec((1, n), lambda i: (0, 0))],
        out_specs=[pl.BlockSpec((bm, n), lambda i: (i, 0)),
                   pl.BlockSpec((1, n), lambda i: (0, 0)),
                   pl.BlockSpec((1, n), lambda i: (0, 0))],
        out_shape=[jax.ShapeDtypeStruct((m, n), jnp.float32),
                   jax.ShapeDtypeStruct((1, n), jnp.float32),
                   jax.ShapeDtypeStruct((1, n), jnp.float32)],
    )(x, w, b.reshape(1, n))


def _mm3_kernel(x_ref, w0, b0, w1, b1, w2, b2, o0, o1, o2):
    x = x_ref[...]
    o0[...] = jnp.dot(x, w0[...], preferred_element_type=jnp.float32,
                          precision=lax.Precision.HIGHEST) + b0[...]
    o1[...] = jnp.dot(x, w1[...], preferred_element_type=jnp.float32,
                          precision=lax.Precision.HIGHEST) + b1[...]
    o2[...] = jnp.dot(x, w2[...], preferred_element_type=jnp.float32,
                          precision=lax.Precision.HIGHEST) + b2[...]


def _mm3(x, wb, bm):
    m, k = x.shape
    n = wb[0][0].shape[1]
    blk = pl.BlockSpec((bm, n), lambda i: (i, 0))
    wspec = pl.BlockSpec((k, n), lambda i: (0, 0))
    bspec = pl.BlockSpec((1, n), lambda i: (0, 0))
    args = [x]
    for w, b in wb:
        args += [w, b.reshape(1, n)]
    return pl.pallas_call(
        _mm3_kernel,
        grid=(m // bm,),
        in_specs=[pl.BlockSpec((bm, k), lambda i: (i, 0))]
        + [wspec, bspec] * 3,
        out_specs=[blk] * 3,
        out_shape=[jax.ShapeDtypeStruct((m, n), jnp.float32)] * 3,
    )(*args)


def _add2_stats_kernel(x_ref, o_ref, s_ref, q_ref):
    y = x_ref[0] + x_ref[1]
    o_ref[...] = y

    @pl.when(pl.program_id(0) == 0)
    def _():
        s_ref[...] = jnp.zeros_like(s_ref)
        q_ref[...] = jnp.zeros_like(q_ref)

    s_ref[...] += jnp.sum(y, axis=0, keepdims=True)
    q_ref[...] += jnp.sum(y * y, axis=0, keepdims=True)


def _add2_stats(parts, bm):
    _, m, n = parts.shape
    return pl.pallas_call(
        _add2_stats_kernel,
        grid=(m // bm,),
        in_specs=[pl.BlockSpec((2, bm, n), lambda i: (0, i, 0))],
        out_specs=[pl.BlockSpec((bm, n), lambda i: (i, 0)),
                   pl.BlockSpec((1, n), lambda i: (0, 0)),
                   pl.BlockSpec((1, n), lambda i: (0, 0))],
        out_shape=[jax.ShapeDtypeStruct((m, n), jnp.float32),
                   jax.ShapeDtypeStruct((1, n), jnp.float32),
                   jax.ShapeDtypeStruct((1, n), jnp.float32)],
    )(parts)


def _pred_kernel(x_ref, w1, b1, w2, b2, o_ref):
    h = x_ref[0] + x_ref[1]
    h = jnp.maximum(
        jnp.dot(h, w1[...], preferred_element_type=jnp.float32,
                          precision=lax.Precision.HIGHEST) + b1[...], 0.0)
    o_ref[...] = jnp.dot(h, w2[...], preferred_element_type=jnp.float32,
                          precision=lax.Precision.HIGHEST) + b2[...]


def _pred(parts, w1, b1, w2p, b2p, bm):
    _, m, n = parts.shape
    wspec = pl.BlockSpec((n, n), lambda i: (0, 0))
    bspec = pl.BlockSpec((1, n), lambda i: (0, 0))
    return pl.pallas_call(
        _pred_kernel,
        grid=(m // bm,),
        in_specs=[pl.BlockSpec((2, bm, n), lambda i: (0, i, 0)),
                  wspec, bspec, wspec, bspec],
        out_specs=pl.BlockSpec((bm, n), lambda i: (i, 0)),
        out_shape=jax.ShapeDtypeStruct((m, n), jnp.float32),
    )(parts, w1, b1.reshape(1, n), w2p, b2p)


@functools.partial(
    pl.kernel,
    out_type=(
        jax.ShapeDtypeStruct((_E,), jnp.float32),
        jax.ShapeDtypeStruct((_NC, _NP), jnp.float32),
    ),
    mesh=_mesh,
    compiler_params=_sc_params,
    scratch_types=dict(
        idx_s0=pltpu.VMEM((_C,), jnp.int32),
        idx_d0=pltpu.VMEM((_C,), jnp.int32),
        ai_v0=pltpu.VMEM((_C, _D), jnp.float32),
        aj_v0=pltpu.VMEM((_C, _D), jnp.float32),
        g_v0=pltpu.VMEM((_C, _D), jnp.float32),
        sem0=pltpu.SemaphoreType.DMA,
        isem0=pltpu.SemaphoreType.DMA,
        idx_s1=pltpu.VMEM((_C,), jnp.int32),
        idx_d1=pltpu.VMEM((_C,), jnp.int32),
        ai_v1=pltpu.VMEM((_C, _D), jnp.float32),
        aj_v1=pltpu.VMEM((_C, _D), jnp.float32),
        g_v1=pltpu.VMEM((_C, _D), jnp.float32),
        sem1=pltpu.SemaphoreType.DMA,
        isem1=pltpu.SemaphoreType.DMA,
        attn_v=pltpu.VMEM((_D,), jnp.float32),
        pbuf=pltpu.VMEM((_C,), jnp.float32),
        stab=pltpu.VMEM((_NP,), jnp.float32),
        acc_c=pltpu.VMEM((_COLS,), jnp.float32),
        tmp_c=pltpu.VMEM((_COLS,), jnp.float32),
        s_sh=pltpu.VMEM_SHARED((_NS, _NP), jnp.float32),
    ),
)
def _sc_alpha(ai_hbm, aj_hbm, g_hbm, src_hbm, dst_hbm, attn_hbm,
              p_hbm, s_hbm,
              idx_s0, idx_d0, ai_v0, aj_v0, g_v0, sem0, isem0,
              idx_s1, idx_d1, ai_v1, aj_v1, g_v1, sem1, isem1,
              attn_v, pbuf, stab, acc_c, tmp_c, s_sh):
    cid = lax.axis_index("c")
    sid = lax.axis_index("s")
    base = (sid * _NC + cid) * _EW

    pltpu.sync_copy(attn_hbm, attn_v)
    z16 = jnp.zeros((16,), jnp.float32)

    def zero_body(i, _):
        stab[pl.ds(i * 16, 16)] = z16
        return 0

    lax.fori_loop(0, _NP // 16, zero_body, 0)

    lane = lax.iota(jnp.int32, 16)

    def bsum(acc):
        for s in (8, 4, 2, 1):
            acc = acc + _shuf(acc, lane ^ s)
        return acc

    bufs = ((idx_s0, idx_d0, ai_v0, aj_v0, g_v0, sem0, isem0),
            (idx_s1, idx_d1, ai_v1, aj_v1, g_v1, sem1, isem1))

    def issue(b, off):
        ids, idd, av, jv, gv, sem, isem = bufs[b]
        pltpu.async_copy(g_hbm.at[pl.ds(off, _C)], gv, sem)
        pltpu.async_copy(src_hbm.at[pl.ds(off, _C)], ids, isem)
        pltpu.async_copy(dst_hbm.at[pl.ds(off, _C)], idd, isem)
        pltpu.make_async_copy(src_hbm.at[pl.ds(0, _C)], ids, isem).wait()
        pltpu.make_async_copy(dst_hbm.at[pl.ds(0, _C)], idd, isem).wait()
        pltpu.async_copy(ai_hbm.at[idd], av, sem)
        pltpu.async_copy(aj_hbm.at[ids], jv, sem)

    def wait(b):
        ids, idd, av, jv, gv, sem, isem = bufs[b]
        pltpu.make_async_copy(ai_hbm.at[idd], av, sem).wait()
        pltpu.make_async_copy(aj_hbm.at[ids], jv, sem).wait()
        pltpu.make_async_copy(g_hbm.at[pl.ds(0, _C)], gv, sem).wait()

    def compute(b, off, ce):
        ids, idd, av, jv, gv, sem, isem = bufs[b]

        def group(g, nothing):
            alv = jnp.zeros((16,), jnp.float32)
            for e16 in range(16):
                e = g * 16 + e16
                acc = jnp.zeros((16,), jnp.float32)
                for t in range(_D // 16):
                    sl = pl.ds(t * 16, 16)
                    v = av[e, sl] + jv[e, sl] + gv[e, sl]
                    v = jnp.where(v > 0, v, 0.01 * v)
                    acc = acc + v * attn_v[sl]
                alv = jnp.where(lane == e16, bsum(acc), alv)
            pv = jnp.exp(alv)
            sl = pl.ds(g * 16, 16)
            pbuf[sl] = pv
            plsc.addupdate_scatter(stab, [idd[sl]], pv)
            return nothing

        lax.fori_loop(0, ce // 16, group, 0)
        pltpu.sync_copy(pbuf.at[pl.ds(0, ce)], p_hbm.at[pl.ds(off, ce)])

    issue(0, base)

    def pair(pi, _):
        off0 = base + (2 * pi) * _C
        issue(1, off0 + _C)
        wait(0)
        compute(0, off0, _C)

        @pl.when(pi < _NCHUNK // 2 - 1)
        def _():
            issue(0, off0 + 2 * _C)

        wait(1)
        compute(1, off0 + _C, _C)
        return 0

    lax.fori_loop(0, _NCHUNK // 2, pair, 0)

    toff = base + _NCHUNK * _C
    pltpu.sync_copy(src_hbm.at[pl.ds(toff, _TAIL)], idx_s0.at[pl.ds(0, _TAIL)])
    pltpu.sync_copy(dst_hbm.at[pl.ds(toff, _TAIL)], idx_d0.at[pl.ds(0, _TAIL)])
    pltpu.async_copy(ai_hbm.at[idx_d0.at[pl.ds(0, _TAIL)]],
                     ai_v0.at[pl.ds(0, _TAIL)], sem0).wait()
    pltpu.async_copy(aj_hbm.at[idx_s0.at[pl.ds(0, _TAIL)]],
                     aj_v0.at[pl.ds(0, _TAIL)], sem0).wait()
    pltpu.sync_copy(g_hbm.at[pl.ds(toff, _TAIL)], g_v0.at[pl.ds(0, _TAIL)])
    compute(0, toff, _TAIL)

    pltpu.sync_copy(stab, s_sh.at[sid])
    plsc.subcore_barrier()
    cols = pl.ds(sid * _COLS, _COLS)
    pltpu.sync_copy(s_sh.at[0, cols], acc_c)

    def merge_body(k, _):
        pltpu.sync_copy(s_sh.at[k, cols], tmp_c)
        for t in range(_COLS // 16):
            sl = pl.ds(t * 16, 16)
            acc_c[sl] = acc_c[sl] + tmp_c[sl]
        return 0

    lax.fori_loop(1, _NS, merge_body, 0)
    pltpu.sync_copy(acc_c, s_hbm.at[cid, cols])


_RB = 128


@functools.partial(
    pl.kernel,
    out_type=jax.ShapeDtypeStruct((_NC, _NP, _D), jnp.float32),
    mesh=_mesh,
    compiler_params=_sc_params,
    scratch_types=dict(
        idx_s0=pltpu.VMEM((_C,), jnp.int32),
        idx_d0=pltpu.VMEM((_C,), jnp.int32),
        rows0=pltpu.VMEM((_C, _D), jnp.float32),
        pb0=pltpu.VMEM((_C,), jnp.float32),
        sem0=pltpu.SemaphoreType.DMA,
        isem0=pltpu.SemaphoreType.DMA,
        idx_s1=pltpu.VMEM((_C,), jnp.int32),
        idx_d1=pltpu.VMEM((_C,), jnp.int32),
        rows1=pltpu.VMEM((_C, _D), jnp.float32),
        pb1=pltpu.VMEM((_C,), jnp.float32),
        sem1=pltpu.SemaphoreType.DMA,
        isem1=pltpu.SemaphoreType.DMA,
        s_tot=pltpu.VMEM((_NP,), jnp.float32),
        s_chk=pltpu.VMEM((_COLS,), jnp.float32),
        acc_sh=pltpu.VMEM_SHARED((_NP, _D), jnp.float32),
    ),
)
def _sc_aggregate(p_hbm, s_hbm, mj_hbm, src_hbm, dst_hbm,
                  out_hbm,
                  idx_s0, idx_d0, rows0, pb0, sem0, isem0,
                  idx_s1, idx_d1, rows1, pb1, sem1, isem1,
                  s_tot, s_chk, acc_sh):
    cid = lax.axis_index("c")
    sid = lax.axis_index("s")
    base = (sid * _NC + cid) * _EW

    pltpu.sync_copy(s_hbm.at[0], s_tot)

    def add_body(k, _):
        pltpu.sync_copy(s_hbm.at[1, pl.ds(k * _COLS, _COLS)], s_chk)
        for t in range(_COLS // 16):
            sl = pl.ds(k * _COLS + t * 16, 16)
            s_tot[sl] = s_tot[sl] + s_chk[pl.ds(t * 16, 16)]
        return 0

    lax.fori_loop(0, _NP // _COLS, add_body, 0)

    z16 = jnp.zeros((16,), jnp.float32)

    def zrow_body(r, _):
        for t in range(_D // 16):
            rows0[r, pl.ds(t * 16, 16)] = z16
        return 0

    lax.fori_loop(0, _RB, zrow_body, 0)
    row0 = sid * (_NP // _NS)
    for i in range(5):
        pltpu.sync_copy(rows0, acc_sh.at[pl.ds(row0 + i * _RB, _RB)])
    plsc.subcore_barrier()

    bufs = ((idx_s0, idx_d0, rows0, pb0, sem0, isem0),
            (idx_s1, idx_d1, rows1, pb1, sem1, isem1))

    def issue(b, off):
        ids, idd, rows, pb, sem, isem = bufs[b]
        pltpu.async_copy(src_hbm.at[pl.ds(off, _C)], ids, isem)
        pltpu.async_copy(dst_hbm.at[pl.ds(off, _C)], idd, isem)
        pltpu.async_copy(p_hbm.at[pl.ds(off, _C)], pb, isem)
        pltpu.make_async_copy(src_hbm.at[pl.ds(0, _C)], ids, isem).wait()
        pltpu.make_async_copy(dst_hbm.at[pl.ds(0, _C)], idd, isem).wait()
        pltpu.make_async_copy(p_hbm.at[pl.ds(0, _C)], pb, isem).wait()
        pltpu.async_copy(mj_hbm.at[ids], rows, sem)

    def wait(b):
        ids, idd, rows, pb, sem, isem = bufs[b]
        pltpu.make_async_copy(mj_hbm.at[ids], rows, sem).wait()

    def compute(b, ce):
        ids, idd, rows, pb, sem, isem = bufs[b]

        def group(g, nothing):
            sl = pl.ds(g * 16, 16)
            sg = plsc.load_gather(s_tot, [idd[sl]])
            av = pb[sl] / (sg + 1e-16)
            for e16 in range(16):
                e = g * 16 + e16
                sv = _shuf(av, jnp.full((16,), e16, jnp.int32))
                for t in range(_D // 16):
                    tsl = pl.ds(t * 16, 16)
                    rows[e, tsl] = rows[e, tsl] * sv
            return nothing

        lax.fori_loop(0, ce // 16, group, 0)
        if ce == _C:
            pltpu.sync_copy(rows, acc_sh.at[idd], add=True)
        else:
            pltpu.sync_copy(rows.at[pl.ds(0, ce)],
                            acc_sh.at[idd.at[pl.ds(0, ce)]], add=True)

    issue(0, base)

    def pair(pi, _):
        off0 = base + (2 * pi) * _C
        issue(1, off0 + _C)
        wait(0)
        compute(0, _C)

        @pl.when(pi < _NCHUNK // 2 - 1)
        def _():
            issue(0, off0 + 2 * _C)

        wait(1)
        compute(1, _C)
        return 0

    lax.fori_loop(0, _NCHUNK // 2, pair, 0)

    toff = base + _NCHUNK * _C
    pltpu.sync_copy(src_hbm.at[pl.ds(toff, _TAIL)], idx_s0.at[pl.ds(0, _TAIL)])
    pltpu.sync_copy(dst_hbm.at[pl.ds(toff, _TAIL)], idx_d0.at[pl.ds(0, _TAIL)])
    pltpu.sync_copy(p_hbm.at[pl.ds(toff, _TAIL)], pb0.at[pl.ds(0, _TAIL)])
    pltpu.async_copy(mj_hbm.at[idx_s0.at[pl.ds(0, _TAIL)]],
                     rows0.at[pl.ds(0, _TAIL)], sem0).wait()
    compute(0, _TAIL)

    plsc.subcore_barrier()
    for i in range(5):
        rsl = pl.ds(row0 + i * _RB, _RB)
        pltpu.sync_copy(acc_sh.at[rsl], rows0)
        pltpu.sync_copy(rows0, out_hbm.at[cid, rsl])


def _affine(g, b, ssum, ssq, m):
    mu = ssum.reshape(-1) / m
    var = ssq.reshape(-1) / m - mu * mu
    a = g / jnp.sqrt(var + _EPS)
    return a, b - a * mu


def _fold(a, c, w, b):
    return a[:, None] * w, b + c @ w


def kernel(x_sta, x_ap, edge_index_r1, edge_attr_r1, edge_index_r2,
           edge_attr_r2, params):
    p = params
    src1 = edge_index_r1[0].astype(jnp.int32)
    dst1 = edge_index_r1[1].astype(jnp.int32)
    src2 = edge_index_r2[0].astype(jnp.int32)
    dst2 = edge_index_r2[1].astype(jnp.int32)

    h_sta, s_sum, s_sq = _mm_stats(x_sta, p["in_sta_W"], p["in_sta_b"], 1000)
    h_ap, a_sum, a_sq = _mm_stats(x_ap, p["in_ap_W"], p["in_ap_b"], 1000)
    e1, e1_sum, e1_sq = _mm_stats(edge_attr_r1, p["ein_r1_W"], p["ein_r1_b"], 2000)
    e2, e2_sum, e2_sq = _mm_stats(edge_attr_r2, p["ein_r2_W"], p["ein_r2_b"], 2000)

    e1_mu = e1_sum.reshape(-1) / _E
    e1_var = e1_sq.reshape(-1) / _E - e1_mu * e1_mu
    e2_mu = e2_sum.reshape(-1) / _E
    e2_var = e2_sq.reshape(-1) / _E - e2_mu * e2_mu

    def edge_affines(mu0, var0):
        out = []
        a_tot = jnp.ones((_D,), jnp.float32)
        c_tot = jnp.zeros((_D,), jnp.float32)
        mu, var = mu0, var0
        for l in range(2):
            g = p["l%d_en_g" % l]
            b = p["l%d_en_b" % l]
            a = g / jnp.sqrt(var + _EPS)
            c = b - a * mu
            a_tot, c_tot = a * a_tot, a * c_tot + c
            out.append((a_tot, c_tot))
            mu, var = b, a * a * var
        return out

    e1_aff = edge_affines(e1_mu, e1_var)
    e2_aff = edge_affines(e2_mu, e2_var)

    node_stats = ((s_sum, s_sq), (a_sum, a_sq))
    out = None
    for l in range(2):
        g_nn, b_nn = p["l%d_nn_g" % l], p["l%d_nn_b" % l]
        sta_a, sta_c = _affine(g_nn, b_nn, *node_stats[0], _N_STA)
        ap_a, ap_c = _affine(g_nn, b_nn, *node_stats[1], _N_AP)
        pr1, pr2 = "l%d_r1" % l, "l%d_r2" % l

        wb_sta = [
            _fold(sta_a, sta_c, p[pr1 + "_ni_W"], p[pr1 + "_ni_b"]),
            _fold(sta_a, sta_c, p[pr2 + "_nj_W"], p[pr2 + "_nj_b"]),
            _fold(sta_a, sta_c, p[pr2 + "_node_W"], p[pr2 + "_node_b"]),
        ]
        wb_ap = [
            _fold(ap_a, ap_c, p[pr2 + "_ni_W"], p[pr2 + "_ni_b"]),
            _fold(ap_a, ap_c, p[pr1 + "_nj_W"], p[pr1 + "_nj_b"]),
            _fold(ap_a, ap_c, p[pr1 + "_node_W"], p[pr1 + "_node_b"]),
        ]
        bm_n = 1000 if l == 0 else 1280
        ai1, aj2, mj2 = _mm3(h_sta, wb_sta, bm_n)
        ai2, aj1, mj1 = _mm3(h_ap, wb_ap, bm_n)

        w_f1, b_f1 = _fold(*e1_aff[l], p[pr1 + "_fij_W"], p[pr1 + "_fij_b"])
        w_f2, b_f2 = _fold(*e2_aff[l], p[pr2 + "_fij_W"], p[pr2 + "_fij_b"])
        g1 = _mm(e1, w_f1, b_f1, 2000)
        g2 = _mm(e2, w_f2, b_f2, 2000)

        p1, seg1 = _sc_alpha(ai1, aj1, g1, src1, dst1,
                             p[pr1 + "_attn"].reshape(-1))
        part1 = _sc_aggregate(p1, seg1, mj1, src1, dst1)
        p2, seg2 = _sc_alpha(ai2, aj2, g2, src2, dst2,
                             p[pr2 + "_attn"].reshape(-1))
        part2 = _sc_aggregate(p2, seg2, mj2, src2, dst2)

        if l == 0:
            h_sta, s_sum, s_sq = _add2_stats(part1, 1280)
            h_ap, a_sum, a_sq = _add2_stats(part2, 1280)
            node_stats = ((s_sum, s_sq), (a_sum, a_sq))
        else:
            w2p = jnp.pad(p["pred_W2"], ((0, 0), (0, _D - 1)))
            b2p = jnp.pad(p["pred_b2"], (0, _D - 1)).reshape(1, _D)
            out = _pred(part1, p["pred_W1"], p["pred_b1"], w2p, b2p, 1280)
    return out[:_N_STA, :1]


# speedup vs baseline: 9.1845x; 1.0612x over previous
import functools

import jax
import jax.numpy as jnp
from jax import lax
from jax.experimental import pallas as pl
from jax.experimental.pallas import tpu as pltpu
from jax.experimental.pallas import tpu_sc as plsc

_N_STA = 10000
_N_AP = 10000
_E = 320000
_D = 128
_EPS = 1e-5

_NC = 2
_NS = 16
_NW = _NC * _NS
_EW = _E // _NW
_C = 128
_NCHUNK = _EW // _C
_TAIL = _EW - _NCHUNK * _C
_NP = 10240
_COLS = _NP // _NS

_mesh = plsc.VectorSubcoreMesh(core_axis_name="c", subcore_axis_name="s")
_sc_params = pltpu.CompilerParams(needs_layout_passes=False)


def _shuf(v, idx):
    dn = lax.GatherDimensionNumbers(offset_dims=(), collapsed_slice_dims=(0,),
                                    start_index_map=(0,))
    return lax.gather(v, idx[:, None], dn, (1,),
                      mode=lax.GatherScatterMode.PROMISE_IN_BOUNDS)


def _mm_kernel(x_ref, w_ref, b_ref, o_ref):
    o_ref[...] = (jnp.dot(x_ref[...], w_ref[...],
                          preferred_element_type=jnp.float32,
                          precision=lax.Precision.HIGHEST) + b_ref[...])


def _mm(x, w, b, bm):
    m, k = x.shape
    n = w.shape[1]
    return pl.pallas_call(
        _mm_kernel,
        grid=(m // bm,),
        in_specs=[pl.BlockSpec((bm, k), lambda i: (i, 0)),
                  pl.BlockSpec((k, n), lambda i: (0, 0)),
                  pl.BlockSpec((1, n), lambda i: (0, 0))],
        out_specs=pl.BlockSpec((bm, n), lambda i: (i, 0)),
        out_shape=jax.ShapeDtypeStruct((m, n), jnp.float32),
    )(x, w, b.reshape(1, n))


def _mm_stats_kernel(x_ref, w_ref, b_ref, o_ref, s_ref, q_ref):
    y = (jnp.dot(x_ref[...], w_ref[...],
                 preferred_element_type=jnp.float32,
                          precision=lax.Precision.HIGHEST) + b_ref[...])
    o_ref[...] = y

    @pl.when(pl.program_id(0) == 0)
    def _():
        s_ref[...] = jnp.zeros_like(s_ref)
        q_ref[...] = jnp.zeros_like(q_ref)

    s_ref[...] += jnp.sum(y, axis=0, keepdims=True)
    q_ref[...] += jnp.sum(y * y, axis=0, keepdims=True)


def _mm_stats(x, w, b, bm):
    m, k = x.shape
    n = w.shape[1]
    return pl.pallas_call(
        _mm_stats_kernel,
        grid=(m // bm,),
        in_specs=[pl.BlockSpec((bm, k), lambda i: (i, 0)),
                  pl.BlockSpec((k, n), lambda i: (0, 0)),
                  pl.BlockSpec((1, n), lambda i: (0, 0))],
        out_specs=[pl.BlockSpec((bm, n), lambda i: (i, 0)),
                   pl.BlockSpec((1, n), lambda i: (0, 0)),
                   pl.BlockSpec((1, n), lambda i: (0, 0))],
        out_shape=[jax.ShapeDtypeStruct((m, n), jnp.float32),
                   jax.ShapeDtypeStruct((1, n), jnp.float32),
                   jax.ShapeDtypeStruct((1, n), jnp.float32)],
    )(x, w, b.reshape(1, n))


def _gram_kernel(x_ref, g_ref, s_ref):
    x = x_ref[...]

    @pl.when(pl.program_id(0) == 0)
    def _():
        g_ref[...] = jnp.zeros_like(g_ref)
        s_ref[...] = jnp.zeros_like(s_ref)

    g_ref[...] += lax.dot_general(x, x, (((0,), (0,)), ((), ())),
                                  preferred_element_type=jnp.float32,
                                  precision=lax.Precision.HIGHEST)
    s_ref[...] += jnp.sum(x, axis=0, keepdims=True)


def _gram(x, bm):
    m, k = x.shape
    return pl.pallas_call(
        _gram_kernel,
        grid=(m // bm,),
        in_specs=[pl.BlockSpec((bm, k), lambda i: (i, 0))],
        out_specs=[pl.BlockSpec((k, k), lambda i: (0, 0)),
                   pl.BlockSpec((1, k), lambda i: (0, 0))],
        out_shape=[jax.ShapeDtypeStruct((k, k), jnp.float32),
                   jax.ShapeDtypeStruct((1, k), jnp.float32)],
    )(x)


def _mm3_kernel(x_ref, w0, b0, w1, b1, w2, b2, o0, o1, o2):
    x = x_ref[...]
    o0[...] = jnp.dot(x, w0[...], preferred_element_type=jnp.float32,
                          precision=lax.Precision.HIGHEST) + b0[...]
    o1[...] = jnp.dot(x, w1[...], preferred_element_type=jnp.float32,
                          precision=lax.Precision.HIGHEST) + b1[...]
    o2[...] = jnp.dot(x, w2[...], preferred_element_type=jnp.float32,
                          precision=lax.Precision.HIGHEST) + b2[...]


def _mm3(x, wb, bm):
    m, k = x.shape
    n = wb[0][0].shape[1]
    blk = pl.BlockSpec((bm, n), lambda i: (i, 0))
    wspec = pl.BlockSpec((k, n), lambda i: (0, 0))
    bspec = pl.BlockSpec((1, n), lambda i: (0, 0))
    args = [x]
    for w, b in wb:
        args += [w, b.reshape(1, n)]
    return pl.pallas_call(
        _mm3_kernel,
        grid=(m // bm,),
        in_specs=[pl.BlockSpec((bm, k), lambda i: (i, 0))]
        + [wspec, bspec] * 3,
        out_specs=[blk] * 3,
        out_shape=[jax.ShapeDtypeStruct((m, n), jnp.float32)] * 3,
    )(*args)


def _add2_stats_kernel(x_ref, o_ref, s_ref, q_ref):
    y = x_ref[0] + x_ref[1]
    o_ref[...] = y

    @pl.when(pl.program_id(0) == 0)
    def _():
        s_ref[...] = jnp.zeros_like(s_ref)
        q_ref[...] = jnp.zeros_like(q_ref)

    s_ref[...] += jnp.sum(y, axis=0, keepdims=True)
    q_ref[...] += jnp.sum(y * y, axis=0, keepdims=True)


def _add2_stats(parts, bm):
    _, m, n = parts.shape
    return pl.pallas_call(
        _add2_stats_kernel,
        grid=(m // bm,),
        in_specs=[pl.BlockSpec((2, bm, n), lambda i: (0, i, 0))],
        out_specs=[pl.BlockSpec((bm, n), lambda i: (i, 0)),
                   pl.BlockSpec((1, n), lambda i: (0, 0)),
                   pl.BlockSpec((1, n), lambda i: (0, 0))],
        out_shape=[jax.ShapeDtypeStruct((m, n), jnp.float32),
                   jax.ShapeDtypeStruct((1, n), jnp.float32),
                   jax.ShapeDtypeStruct((1, n), jnp.float32)],
    )(parts)


def _pred_kernel(x_ref, w1, b1, w2, b2, o_ref):
    h = x_ref[0] + x_ref[1]
    h = jnp.maximum(
        jnp.dot(h, w1[...], preferred_element_type=jnp.float32,
                          precision=lax.Precision.HIGHEST) + b1[...], 0.0)
    o_ref[...] = jnp.dot(h, w2[...], preferred_element_type=jnp.float32,
                          precision=lax.Precision.HIGHEST) + b2[...]


def _pred(parts, w1, b1, w2p, b2p, bm):
    _, m, n = parts.shape
    wspec = pl.BlockSpec((n, n), lambda i: (0, 0))
    bspec = pl.BlockSpec((1, n), lambda i: (0, 0))
    return pl.pallas_call(
        _pred_kernel,
        grid=(m // bm,),
        in_specs=[pl.BlockSpec((2, bm, n), lambda i: (0, i, 0)),
                  wspec, bspec, wspec, bspec],
        out_specs=pl.BlockSpec((bm, n), lambda i: (i, 0)),
        out_shape=jax.ShapeDtypeStruct((m, n), jnp.float32),
    )(parts, w1, b1.reshape(1, n), w2p, b2p)


@functools.partial(
    pl.kernel,
    out_type=(
        jax.ShapeDtypeStruct((_E,), jnp.float32),
        jax.ShapeDtypeStruct((_NC, _NP), jnp.float32),
    ),
    mesh=_mesh,
    compiler_params=_sc_params,
    scratch_types=dict(
        idx_s0=pltpu.VMEM((_C,), jnp.int32),
        idx_d0=pltpu.VMEM((_C,), jnp.int32),
        ai_v0=pltpu.VMEM((_C, _D), jnp.float32),
        aj_v0=pltpu.VMEM((_C, _D), jnp.float32),
        g_v0=pltpu.VMEM((_C, _D), jnp.float32),
        sem0=pltpu.SemaphoreType.DMA,
        isem0=pltpu.SemaphoreType.DMA,
        idx_s1=pltpu.VMEM((_C,), jnp.int32),
        idx_d1=pltpu.VMEM((_C,), jnp.int32),
        ai_v1=pltpu.VMEM((_C, _D), jnp.float32),
        aj_v1=pltpu.VMEM((_C, _D), jnp.float32),
        g_v1=pltpu.VMEM((_C, _D), jnp.float32),
        sem1=pltpu.SemaphoreType.DMA,
        isem1=pltpu.SemaphoreType.DMA,
        attn_v=pltpu.VMEM((_D,), jnp.float32),
        pbuf=pltpu.VMEM((_C,), jnp.float32),
        stab=pltpu.VMEM((_NP,), jnp.float32),
        acc_c=pltpu.VMEM((_COLS,), jnp.float32),
        tmp_c=pltpu.VMEM((_COLS,), jnp.float32),
        s_sh=pltpu.VMEM_SHARED((_NS, _NP), jnp.float32),
    ),
)
def _sc_alpha(ai_hbm, aj_hbm, g_hbm, src_hbm, dst_hbm, attn_hbm,
              p_hbm, s_hbm,
              idx_s0, idx_d0, ai_v0, aj_v0, g_v0, sem0, isem0,
              idx_s1, idx_d1, ai_v1, aj_v1, g_v1, sem1, isem1,
              attn_v, pbuf, stab, acc_c, tmp_c, s_sh):
    cid = lax.axis_index("c")
    sid = lax.axis_index("s")
    base = (sid * _NC + cid) * _EW

    pltpu.sync_copy(attn_hbm, attn_v)
    z16 = jnp.zeros((16,), jnp.float32)

    def zero_body(i, _):
        stab[pl.ds(i * 16, 16)] = z16
        return 0

    lax.fori_loop(0, _NP // 16, zero_body, 0)

    lane = lax.iota(jnp.int32, 16)

    def bsum(acc):
        for s in (8, 4, 2, 1):
            acc = acc + _shuf(acc, lane ^ s)
        return acc

    bufs = ((idx_s0, idx_d0, ai_v0, aj_v0, g_v0, sem0, isem0),
            (idx_s1, idx_d1, ai_v1, aj_v1, g_v1, sem1, isem1))

    def issue(b, off):
        ids, idd, av, jv, gv, sem, isem = bufs[b]
        pltpu.async_copy(g_hbm.at[pl.ds(off, _C)], gv, sem)
        pltpu.async_copy(src_hbm.at[pl.ds(off, _C)], ids, isem)
        pltpu.async_copy(dst_hbm.at[pl.ds(off, _C)], idd, isem)
        pltpu.make_async_copy(src_hbm.at[pl.ds(0, _C)], ids, isem).wait()
        pltpu.make_async_copy(dst_hbm.at[pl.ds(0, _C)], idd, isem).wait()
        pltpu.async_copy(ai_hbm.at[idd], av, sem)
        pltpu.async_copy(aj_hbm.at[ids], jv, sem)

    def wait(b):
        ids, idd, av, jv, gv, sem, isem = bufs[b]
        pltpu.make_async_copy(ai_hbm.at[idd], av, sem).wait()
        pltpu.make_async_copy(aj_hbm.at[ids], jv, sem).wait()
        pltpu.make_async_copy(g_hbm.at[pl.ds(0, _C)], gv, sem).wait()

    def compute(b, off, ce):
        ids, idd, av, jv, gv, sem, isem = bufs[b]

        def group(g, nothing):
            alv = jnp.zeros((16,), jnp.float32)
            for e16 in range(16):
                e = g * 16 + e16
                acc = jnp.zeros((16,), jnp.float32)
                for t in range(_D // 16):
                    sl = pl.ds(t * 16, 16)
                    v = av[e, sl] + jv[e, sl] + gv[e, sl]
                    v = jnp.where(v > 0, v, 0.01 * v)
                    acc = acc + v * attn_v[sl]
                alv = jnp.where(lane == e16, bsum(acc), alv)
            pv = jnp.exp(alv)
            sl = pl.ds(g * 16, 16)
            pbuf[sl] = pv
            plsc.addupdate_scatter(stab, [idd[sl]], pv)
            return nothing

        lax.fori_loop(0, ce // 16, group, 0)
        pltpu.sync_copy(pbuf.at[pl.ds(0, ce)], p_hbm.at[pl.ds(off, ce)])

    issue(0, base)

    def pair(pi, _):
        off0 = base + (2 * pi) * _C
        issue(1, off0 + _C)
        wait(0)
        compute(0, off0, _C)

        @pl.when(pi < _NCHUNK // 2 - 1)
        def _():
            issue(0, off0 + 2 * _C)

        wait(1)
        compute(1, off0 + _C, _C)
        return 0

    lax.fori_loop(0, _NCHUNK // 2, pair, 0)

    toff = base + _NCHUNK * _C
    pltpu.sync_copy(src_hbm.at[pl.ds(toff, _TAIL)], idx_s0.at[pl.ds(0, _TAIL)])
    pltpu.sync_copy(dst_hbm.at[pl.ds(toff, _TAIL)], idx_d0.at[pl.ds(0, _TAIL)])
    pltpu.async_copy(ai_hbm.at[idx_d0.at[pl.ds(0, _TAIL)]],
                     ai_v0.at[pl.ds(0, _TAIL)], sem0).wait()
    pltpu.async_copy(aj_hbm.at[idx_s0.at[pl.ds(0, _TAIL)]],
                     aj_v0.at[pl.ds(0, _TAIL)], sem0).wait()
    pltpu.sync_copy(g_hbm.at[pl.ds(toff, _TAIL)], g_v0.at[pl.ds(0, _TAIL)])
    compute(0, toff, _TAIL)

    pltpu.sync_copy(stab, s_sh.at[sid])
    plsc.subcore_barrier()
    cols = pl.ds(sid * _COLS, _COLS)
    pltpu.sync_copy(s_sh.at[0, cols], acc_c)

    def merge_body(k, _):
        pltpu.sync_copy(s_sh.at[k, cols], tmp_c)
        for t in range(_COLS // 16):
            sl = pl.ds(t * 16, 16)
            acc_c[sl] = acc_c[sl] + tmp_c[sl]
        return 0

    lax.fori_loop(1, _NS, merge_body, 0)
    pltpu.sync_copy(acc_c, s_hbm.at[cid, cols])


_RB = 128


@functools.partial(
    pl.kernel,
    out_type=jax.ShapeDtypeStruct((_NC, _NP, _D), jnp.float32),
    mesh=_mesh,
    compiler_params=_sc_params,
    scratch_types=dict(
        idx_s0=pltpu.VMEM((_C,), jnp.int32),
        idx_d0=pltpu.VMEM((_C,), jnp.int32),
        rows0=pltpu.VMEM((_C, _D), jnp.float32),
        pb0=pltpu.VMEM((_C,), jnp.float32),
        sem0=pltpu.SemaphoreType.DMA,
        isem0=pltpu.SemaphoreType.DMA,
        idx_s1=pltpu.VMEM((_C,), jnp.int32),
        idx_d1=pltpu.VMEM((_C,), jnp.int32),
        rows1=pltpu.VMEM((_C, _D), jnp.float32),
        pb1=pltpu.VMEM((_C,), jnp.float32),
        sem1=pltpu.SemaphoreType.DMA,
        isem1=pltpu.SemaphoreType.DMA,
        s_tot=pltpu.VMEM((_NP,), jnp.float32),
        s_chk=pltpu.VMEM((_COLS,), jnp.float32),
        acc_sh=pltpu.VMEM_SHARED((_NP, _D), jnp.float32),
    ),
)
def _sc_aggregate(p_hbm, s_hbm, mj_hbm, src_hbm, dst_hbm,
                  out_hbm,
                  idx_s0, idx_d0, rows0, pb0, sem0, isem0,
                  idx_s1, idx_d1, rows1, pb1, sem1, isem1,
                  s_tot, s_chk, acc_sh):
    cid = lax.axis_index("c")
    sid = lax.axis_index("s")
    base = (sid * _NC + cid) * _EW

    pltpu.sync_copy(s_hbm.at[0], s_tot)

    def add_body(k, _):
        pltpu.sync_copy(s_hbm.at[1, pl.ds(k * _COLS, _COLS)], s_chk)
        for t in range(_COLS // 16):
            sl = pl.ds(k * _COLS + t * 16, 16)
            s_tot[sl] = s_tot[sl] + s_chk[pl.ds(t * 16, 16)]
        return 0

    lax.fori_loop(0, _NP // _COLS, add_body, 0)

    z16 = jnp.zeros((16,), jnp.float32)

    def zrow_body(r, _):
        for t in range(_D // 16):
            rows0[r, pl.ds(t * 16, 16)] = z16
        return 0

    lax.fori_loop(0, _RB, zrow_body, 0)
    row0 = sid * (_NP // _NS)
    for i in range(5):
        pltpu.sync_copy(rows0, acc_sh.at[pl.ds(row0 + i * _RB, _RB)])
    plsc.subcore_barrier()

    bufs = ((idx_s0, idx_d0, rows0, pb0, sem0, isem0),
            (idx_s1, idx_d1, rows1, pb1, sem1, isem1))

    def issue(b, off):
        ids, idd, rows, pb, sem, isem = bufs[b]
        pltpu.async_copy(src_hbm.at[pl.ds(off, _C)], ids, isem)
        pltpu.async_copy(dst_hbm.at[pl.ds(off, _C)], idd, isem)
        pltpu.async_copy(p_hbm.at[pl.ds(off, _C)], pb, isem)
        pltpu.make_async_copy(src_hbm.at[pl.ds(0, _C)], ids, isem).wait()
        pltpu.make_async_copy(dst_hbm.at[pl.ds(0, _C)], idd, isem).wait()
        pltpu.make_async_copy(p_hbm.at[pl.ds(0, _C)], pb, isem).wait()
        pltpu.async_copy(mj_hbm.at[ids], rows, sem)

    def wait(b):
        ids, idd, rows, pb, sem, isem = bufs[b]
        pltpu.make_async_copy(mj_hbm.at[ids], rows, sem).wait()

    def compute(b, ce):
        ids, idd, rows, pb, sem, isem = bufs[b]

        def group(g, nothing):
            sl = pl.ds(g * 16, 16)
            sg = plsc.load_gather(s_tot, [idd[sl]])
            av = pb[sl] / (sg + 1e-16)
            for e16 in range(16):
                e = g * 16 + e16
                sv = _shuf(av, jnp.full((16,), e16, jnp.int32))
                for t in range(_D // 16):
                    tsl = pl.ds(t * 16, 16)
                    rows[e, tsl] = rows[e, tsl] * sv
            return nothing

        lax.fori_loop(0, ce // 16, group, 0)
        if ce == _C:
            pltpu.sync_copy(rows, acc_sh.at[idd], add=True)
        else:
            pltpu.sync_copy(rows.at[pl.ds(0, ce)],
                            acc_sh.at[idd.at[pl.ds(0, ce)]], add=True)

    issue(0, base)

    def pair(pi, _):
        off0 = base + (2 * pi) * _C
        issue(1, off0 + _C)
        wait(0)
        compute(0, _C)

        @pl.when(pi < _NCHUNK // 2 - 1)
        def _():
            issue(0, off0 + 2 * _C)

        wait(1)
        compute(1, _C)
        return 0

    lax.fori_loop(0, _NCHUNK // 2, pair, 0)

    toff = base + _NCHUNK * _C
    pltpu.sync_copy(src_hbm.at[pl.ds(toff, _TAIL)], idx_s0.at[pl.ds(0, _TAIL)])
    pltpu.sync_copy(dst_hbm.at[pl.ds(toff, _TAIL)], idx_d0.at[pl.ds(0, _TAIL)])
    pltpu.sync_copy(p_hbm.at[pl.ds(toff, _TAIL)], pb0.at[pl.ds(0, _TAIL)])
    pltpu.async_copy(mj_hbm.at[idx_s0.at[pl.ds(0, _TAIL)]],
                     rows0.at[pl.ds(0, _TAIL)], sem0).wait()
    compute(0, _TAIL)

    plsc.subcore_barrier()
    for i in range(5):
        rsl = pl.ds(row0 + i * _RB, _RB)
        pltpu.sync_copy(acc_sh.at[rsl], rows0)
        pltpu.sync_copy(rows0, out_hbm.at[cid, rsl])


def _affine(g, b, ssum, ssq, m):
    mu = ssum.reshape(-1) / m
    var = ssq.reshape(-1) / m - mu * mu
    a = g / jnp.sqrt(var + _EPS)
    return a, b - a * mu


def _fold(a, c, w, b):
    return a[:, None] * w, b + c @ w


def kernel(x_sta, x_ap, edge_index_r1, edge_attr_r1, edge_index_r2,
           edge_attr_r2, params):
    p = params
    src1 = edge_index_r1[0].astype(jnp.int32)
    dst1 = edge_index_r1[1].astype(jnp.int32)
    src2 = edge_index_r2[0].astype(jnp.int32)
    dst2 = edge_index_r2[1].astype(jnp.int32)

    h_sta, s_sum, s_sq = _mm_stats(x_sta, p["in_sta_W"], p["in_sta_b"], 1000)
    h_ap, a_sum, a_sq = _mm_stats(x_ap, p["in_ap_W"], p["in_ap_b"], 1000)
    g1m, cs1 = _gram(edge_attr_r1, 2000)
    g2m, cs2 = _gram(edge_attr_r2, 2000)

    def egram_stats(gm, cs, w, b):
        m1 = (cs.reshape(-1) @ w) / _E
        mu = m1 + b
        ex2 = jnp.sum(w * (gm @ w), axis=0) / _E + 2.0 * b * m1 + b * b
        return mu, ex2 - mu * mu

    e1_mu, e1_var = egram_stats(g1m, cs1, p["ein_r1_W"], p["ein_r1_b"])
    e2_mu, e2_var = egram_stats(g2m, cs2, p["ein_r2_W"], p["ein_r2_b"])

    def edge_affines(mu0, var0):
        out = []
        a_tot = jnp.ones((_D,), jnp.float32)
        c_tot = jnp.zeros((_D,), jnp.float32)
        mu, var = mu0, var0
        for l in range(2):
            g = p["l%d_en_g" % l]
            b = p["l%d_en_b" % l]
            a = g / jnp.sqrt(var + _EPS)
            c = b - a * mu
            a_tot, c_tot = a * a_tot, a * c_tot + c
            out.append((a_tot, c_tot))
            mu, var = b, a * a * var
        return out

    e1_aff = edge_affines(e1_mu, e1_var)
    e2_aff = edge_affines(e2_mu, e2_var)

    node_stats = ((s_sum, s_sq), (a_sum, a_sq))
    out = None
    for l in range(2):
        g_nn, b_nn = p["l%d_nn_g" % l], p["l%d_nn_b" % l]
        sta_a, sta_c = _affine(g_nn, b_nn, *node_stats[0], _N_STA)
        ap_a, ap_c = _affine(g_nn, b_nn, *node_stats[1], _N_AP)
        pr1, pr2 = "l%d_r1" % l, "l%d_r2" % l

        wb_sta = [
            _fold(sta_a, sta_c, p[pr1 + "_ni_W"], p[pr1 + "_ni_b"]),
            _fold(sta_a, sta_c, p[pr2 + "_nj_W"], p[pr2 + "_nj_b"]),
            _fold(sta_a, sta_c, p[pr2 + "_node_W"], p[pr2 + "_node_b"]),
        ]
        wb_ap = [
            _fold(ap_a, ap_c, p[pr2 + "_ni_W"], p[pr2 + "_ni_b"]),
            _fold(ap_a, ap_c, p[pr1 + "_nj_W"], p[pr1 + "_nj_b"]),
            _fold(ap_a, ap_c, p[pr1 + "_node_W"], p[pr1 + "_node_b"]),
        ]
        bm_n = 1000 if l == 0 else 1280
        ai1, aj2, mj2 = _mm3(h_sta, wb_sta, bm_n)
        ai2, aj1, mj1 = _mm3(h_ap, wb_ap, bm_n)

        w_f1, b_f1 = _fold(*e1_aff[l], p[pr1 + "_fij_W"], p[pr1 + "_fij_b"])
        w_f2, b_f2 = _fold(*e2_aff[l], p[pr2 + "_fij_W"], p[pr2 + "_fij_b"])
        wc1 = p["ein_r1_W"] @ w_f1
        bc1 = p["ein_r1_b"] @ w_f1 + b_f1
        wc2 = p["ein_r2_W"] @ w_f2
        bc2 = p["ein_r2_b"] @ w_f2 + b_f2
        g1 = _mm(edge_attr_r1, wc1, bc1, 2000)
        g2 = _mm(edge_attr_r2, wc2, bc2, 2000)

        p1, seg1 = _sc_alpha(ai1, aj1, g1, src1, dst1,
                             p[pr1 + "_attn"].reshape(-1))
        part1 = _sc_aggregate(p1, seg1, mj1, src1, dst1)
        p2, seg2 = _sc_alpha(ai2, aj2, g2, src2, dst2,
                             p[pr2 + "_attn"].reshape(-1))
        part2 = _sc_aggregate(p2, seg2, mj2, src2, dst2)

        if l == 0:
            h_sta, s_sum, s_sq = _add2_stats(part1, 1280)
            h_ap, a_sum, a_sq = _add2_stats(part2, 1280)
            node_stats = ((s_sum, s_sq), (a_sum, a_sq))
        else:
            w2p = jnp.pad(p["pred_W2"], ((0, 0), (0, _D - 1)))
            b2p = jnp.pad(p["pred_b2"], (0, _D - 1)).reshape(1, _D)
            out = _pred(part1, p["pred_W1"], p["pred_b1"], w2p, b2p, 1280)
    return out[:_N_STA, :1]
